# Initial kernel scaffold; baseline (speedup 1.0000x reference)
import functools

import jax
import jax.numpy as jnp
from jax import lax
from jax.experimental import pallas as pl
from jax.experimental.pallas import tpu as pltpu
from jax.experimental.pallas import tpu_sc as plsc

NB = 16
KP = 30
NN = 10000
NE = 320000
DF = 128
NP = 10240
DUMMY = 10200
NC, NS, L = 2, 16, 16
CH = 128
EPT_P = 20480
EP = EPT_P * NS
BR = 1024


def _sc_scatter(W):
    RT = NP // NS
    RB = 320
    nch = EPT_P // CH
    mesh = plsc.VectorSubcoreMesh(core_axis_name="c", subcore_axis_name="s")

    def body(hp_t, src_t, dst_t, hp_l, src_l, dst_l, out, sidx, didx,
             rows, zb, acc, sem):
        c = lax.axis_index("c")
        s = lax.axis_index("s")

        def zloop(i, _):
            r = i // (W // L)
            q = i % (W // L)
            zb[r, pl.ds(q * L, L)] = jnp.zeros((L,), jnp.float32)
            return 0
        lax.fori_loop(0, RB * (W // L), zloop, 0)
        for t in range(RT // RB):
            pltpu.sync_copy(zb, acc.at[pl.ds(s * RT + t * RB, RB)])
        plsc.subcore_barrier()

        def run(hp, srce, dste):
            def chunk(j, _):
                base = s * EPT_P + j * CH
                pltpu.sync_copy(srce.at[pl.ds(base, CH)], sidx)
                pltpu.sync_copy(dste.at[pl.ds(base, CH)], didx)
                pltpu.async_copy(hp.at[sidx], rows, sem).wait()
                pltpu.sync_copy(rows, acc.at[didx], add=True)
                return 0
            lax.fori_loop(0, nch, chunk, 0)

        @pl.when(c == 0)
        def _():
            run(hp_t, src_t, dst_t)

        @pl.when(c == 1)
        def _():
            run(hp_l, src_l, dst_l)

        plsc.subcore_barrier()
        for t in range(RT // RB):
            r0 = s * RT + t * RB
            pltpu.sync_copy(acc.at[pl.ds(r0, RB)], zb)
            pltpu.sync_copy(zb, out.at[c, pl.ds(r0, RB)])

    return pl.kernel(
        body,
        out_type=jax.ShapeDtypeStruct((NC, NP, W), jnp.float32),
        mesh=mesh,
        scratch_types=[
            pltpu.VMEM((CH,), jnp.int32),
            pltpu.VMEM((CH,), jnp.int32),
            pltpu.VMEM((CH, W), jnp.float32),
            pltpu.VMEM((RB, W), jnp.float32),
            pltpu.VMEM_SHARED((NP, W), jnp.float32),
            pltpu.SemaphoreType.DMA,
        ],
    )


def _sortpool():
    NV = NN // L
    mesh = plsc.VectorSubcoreMesh(core_axis_name="c", subcore_axis_name="s")

    def body(y_t, lc_t, bt_t, y_l, lc_l, bt_l, out_t, out_l,
             btv, lcv, vals, posb, isel, rows, sem):
        c = lax.axis_index("c")
        s = lax.axis_index("s")
        lanes = lax.iota(jnp.int32, L)
        neg = jnp.full((L,), -jnp.inf, jnp.float32)

        def run(y, lc, bt, out):
            b = s
            pltpu.sync_copy(bt, btv)
            pltpu.sync_copy(lc.at[pl.ds(0, NN)], lcv)

            def pre(i, _):
                vals[pl.ds(i * L, L)] = neg
                posb[pl.ds(i * L, L)] = jnp.full((L,), DUMMY, jnp.int32)
                return 0
            lax.fori_loop(0, NP // L, pre, 0)
            isel[pl.ds(0, L)] = jnp.full((L,), DUMMY, jnp.int32)
            isel[pl.ds(L, L)] = jnp.full((L,), DUMMY, jnp.int32)

            def comp(k2, cnt):
                m = btv[pl.ds(k2 * L, L)] == b
                v = lcv[pl.ds(k2 * L, L)]
                plsc.store_compressed(vals.at[pl.ds(cnt, L)], v, m)
                plsc.store_compressed(posb.at[pl.ds(cnt, L)], lanes + k2 * L, m)
                return cnt + jnp.max(plsc.all_reduce_population_count(m))
            cnt = lax.fori_loop(0, NV, comp, jnp.int32(0))
            nvec = (cnt + L - 1) // L

            def sel(t, _):
                mv = lax.fori_loop(
                    0, nvec,
                    lambda j, a: jnp.maximum(a, vals[pl.ds(j * L, L)]), neg)
                mx = jnp.max(mv)
                j = lax.while_loop(
                    lambda j: jnp.logical_not(
                        jnp.any(vals[pl.ds(j * L, L)] == mx)),
                    lambda j: j + 1, jnp.int32(0))
                v = vals[pl.ds(j * L, L)]
                eq = v == mx
                first = jnp.logical_and(eq, lanes == plsc.all_reduce_ffs(eq))
                pos = posb[pl.ds(j * L, L)]
                posx = jnp.where(mx == -jnp.inf,
                                 jnp.full((L,), DUMMY, jnp.int32), pos)
                plsc.store_scatter(isel, [jnp.full((L,), t, jnp.int32)],
                                   posx, mask=first)
                vals[pl.ds(j * L, L)] = jnp.where(first, neg, v)
                return 0
            lax.fori_loop(0, KP, sel, 0)

            pltpu.async_copy(y.at[isel], rows, sem).wait()
            pltpu.sync_copy(rows.at[pl.ds(0, KP)], out.at[b])

        @pl.when(c == 0)
        def _():
            run(y_t, lc_t, bt_t, out_t)

        @pl.when(c == 1)
        def _():
            run(y_l, lc_l, bt_l, out_l)

    return pl.kernel(
        body,
        out_type=(jax.ShapeDtypeStruct((NB, KP, 32), jnp.float32),
                  jax.ShapeDtypeStruct((NB, KP, 32), jnp.float32)),
        mesh=mesh,
        scratch_types=[
            pltpu.VMEM((NN,), jnp.int32),
            pltpu.VMEM((NN,), jnp.float32),
            pltpu.VMEM((NP,), jnp.float32),
            pltpu.VMEM((NP,), jnp.int32),
            pltpu.VMEM((2 * L,), jnp.int32),
            pltpu.VMEM((2 * L, 32), jnp.float32),
            pltpu.SemaphoreType.DMA,
        ],
    )


def _dinv():
    def k(pref, oref):
        oref[0, :] = lax.rsqrt(pref[0, :, 0] + 1.0)
    return pl.pallas_call(
        k, grid=(2, NP // BR),
        in_specs=[pl.BlockSpec((1, BR, 16), lambda b, i: (b, i, 0))],
        out_specs=pl.BlockSpec((1, BR), lambda b, i: (b, i)),
        out_shape=jax.ShapeDtypeStruct((2, NP), jnp.float32))


def _mm1():
    def k(xref, wref, dref, oref):
        h = jnp.dot(xref[...], wref[...], preferred_element_type=jnp.float32)
        oref[...] = dref[...][:, None] * h
    return pl.pallas_call(
        k, grid=(NP // BR,),
        in_specs=[pl.BlockSpec((BR, DF), lambda i: (i, 0)),
                  pl.BlockSpec((DF, 64), lambda i: (0, 0)),
                  pl.BlockSpec((BR,), lambda i: (i,))],
        out_specs=pl.BlockSpec((BR, 64), lambda i: (i, 0)),
        out_shape=jax.ShapeDtypeStruct((NP, 64), jnp.float32))


def _layer(b, W, Wo):
    def k(sref, href, dref, b1ref, lwref, lbref, wnref, oref):
        dv = dref[...][:, None]
        g = dv * (sref[0] + href[...]) + b1ref[...][None, :]
        x2 = jnp.where(g >= 0, g, 0.01 * g) + jnp.dot(
            g, lwref[...], preferred_element_type=jnp.float32) + lbref[...][None, :]
        oref[...] = dv * jnp.dot(x2, wnref[...],
                                 preferred_element_type=jnp.float32)
    return pl.pallas_call(
        k, grid=(NP // BR,),
        in_specs=[pl.BlockSpec((1, BR, W), lambda i: (b, i, 0)),
                  pl.BlockSpec((BR, W), lambda i: (i, 0)),
                  pl.BlockSpec((BR,), lambda i: (i,)),
                  pl.BlockSpec((W,), lambda i: (0,)),
                  pl.BlockSpec((W, W), lambda i: (0, 0)),
                  pl.BlockSpec((W,), lambda i: (0,)),
                  pl.BlockSpec((W, Wo), lambda i: (0, 0))],
        out_specs=pl.BlockSpec((BR, Wo), lambda i: (i, 0)),
        out_shape=jax.ShapeDtypeStruct((NP, Wo), jnp.float32))


def _final(b):
    def k(sref, href, dref, boref, yref, lref):
        dv = dref[...][:, None]
        g = dv * (sref[0] + href[...]) + boref[...][None, :]
        rid = lax.broadcasted_iota(jnp.int32, (BR, 1), 0) + pl.program_id(0) * BR
        y = jnp.where(rid < NN, g, 0.0)
        yref[...] = y
        lref[...] = y[:, 31]
    return pl.pallas_call(
        k, grid=(NP // BR,),
        in_specs=[pl.BlockSpec((1, BR, 32), lambda i: (b, i, 0)),
                  pl.BlockSpec((BR, 32), lambda i: (i, 0)),
                  pl.BlockSpec((BR,), lambda i: (i,)),
                  pl.BlockSpec((32,), lambda i: (0,))],
        out_specs=[pl.BlockSpec((BR, 32), lambda i: (i, 0)),
                   pl.BlockSpec((BR,), lambda i: (i,))],
        out_shape=[jax.ShapeDtypeStruct((NP, 32), jnp.float32),
                   jax.ShapeDtypeStruct((NP,), jnp.float32)])


def _mlp():
    def k(xt, xl, w1, b1, w2, b2, wo, bo, oref):
        w1f = w1[...]
        a = (jnp.dot(xt[...], w1f[:KP * 32], preferred_element_type=jnp.float32)
             + jnp.dot(xl[...], w1f[KP * 32:], preferred_element_type=jnp.float32)
             + b1[...][None, :])
        a = jnp.where(a >= 0, a, 0.01 * a)
        h = jnp.dot(a, w2[...], preferred_element_type=jnp.float32) + b2[...][None, :]
        h = jnp.where(h >= 0, h, 0.01 * h)
        oref[...] = jnp.dot(h, wo[...], preferred_element_type=jnp.float32) + bo[...][None, :]
    return pl.pallas_call(
        k, out_shape=jax.ShapeDtypeStruct((NB, 1), jnp.float32))


_scat16 = _sc_scatter(16)
_scat64 = _sc_scatter(64)
_scat32 = _sc_scatter(32)
_sortp = _sortpool()
_dinv_k = _dinv()
_mm1_k = _mm1()
_lay1t = _layer(0, 64, 64)
_lay1l = _layer(1, 64, 64)
_lay2t = _layer(0, 64, 32)
_lay2l = _layer(1, 64, 32)
_fin_t = _final(0)
_fin_l = _final(1)
_mlp_k = _mlp()


def _pad_edges(e):
    return jnp.pad(e, (0, EP - NE), constant_values=DUMMY)


def kernel(x_topo, edge_index_topo, x_topo_batch, x_lc, edge_index_lc,
           x_lc_batch, topo_params, lc_params, mlp_params):
    f32 = jnp.float32
    xt = jnp.pad(x_topo.astype(f32), ((0, NP - NN), (0, 0)))
    xl = jnp.pad(x_lc.astype(f32), ((0, NP - NN), (0, 0)))
    src_t = _pad_edges(edge_index_topo[0])
    dst_t = _pad_edges(edge_index_topo[1])
    src_l = _pad_edges(edge_index_lc[0])
    dst_l = _pad_edges(edge_index_lc[1])
    ones = jnp.ones((NP, 16), f32)
    tp, lp = topo_params, lc_params

    degp = _scat16(ones, dst_t, dst_t, ones, dst_l, dst_l)
    dinv2 = _dinv_k(degp)
    dvt, dvl = dinv2[0], dinv2[1]

    hp1t = _mm1_k(xt, tp[0], dvt)
    hp1l = _mm1_k(xl, lp[0], dvl)
    S1 = _scat64(hp1t, src_t, dst_t, hp1l, src_l, dst_l)

    hp2t = _lay1t(S1, hp1t, dvt, tp[1], tp[2], tp[3], tp[4])
    hp2l = _lay1l(S1, hp1l, dvl, lp[1], lp[2], lp[3], lp[4])
    S2 = _scat64(hp2t, src_t, dst_t, hp2l, src_l, dst_l)

    hp3t = _lay2t(S2, hp2t, dvt, tp[5], tp[6], tp[7], tp[8])
    hp3l = _lay2l(S2, hp2l, dvl, lp[5], lp[6], lp[7], lp[8])
    S3 = _scat32(hp3t, src_t, dst_t, hp3l, src_l, dst_l)

    yt, lct = _fin_t(S3, hp3t, dvt, tp[9])
    yl, lcl = _fin_l(S3, hp3l, dvl, lp[9])

    pt, plc = _sortp(yt, lct, x_topo_batch, yl, lcl, x_lc_batch)
    mW1, mb1, mW2, mb2, mWo, mbo = mlp_params
    return _mlp_k(pt.reshape(NB, KP * 32), plc.reshape(NB, KP * 32),
                  mW1, mb1, mW2, mb2, mWo, mbo)


# speedup vs baseline: 9.5062x; 9.5062x over previous
import functools

import jax
import jax.numpy as jnp
from jax import lax
from jax.experimental import pallas as pl
from jax.experimental.pallas import tpu as pltpu
from jax.experimental.pallas import tpu_sc as plsc

NB = 16
KP = 30
NN = 10000
NE = 320000
DF = 128
NP = 10240
DUMMY = 10200
NC, NS, L = 2, 16, 16
CH = 128
EPT_P = 20480
EP = EPT_P * NS
BR = 1024


def _sc_scatter(W):
    RT = NP // NS
    RB = 320
    nch = EPT_P // CH
    mesh = plsc.VectorSubcoreMesh(core_axis_name="c", subcore_axis_name="s")

    def body(hp_t, src_t, dst_t, hp_l, src_l, dst_l, out, sidx, didx,
             rows, zb, acc, sem):
        c = lax.axis_index("c")
        s = lax.axis_index("s")

        def zloop(i, _):
            r = i // (W // L)
            q = i % (W // L)
            zb[r, pl.ds(q * L, L)] = jnp.zeros((L,), jnp.float32)
            return 0
        lax.fori_loop(0, RB * (W // L), zloop, 0)
        for t in range(RT // RB):
            pltpu.sync_copy(zb, acc.at[pl.ds(s * RT + t * RB, RB)])
        plsc.subcore_barrier()

        def run(hp, srce, dste):
            def chunk(j, _):
                base = s * EPT_P + j * CH
                pltpu.sync_copy(srce.at[pl.ds(base, CH)], sidx)
                pltpu.sync_copy(dste.at[pl.ds(base, CH)], didx)
                pltpu.async_copy(hp.at[sidx], rows, sem).wait()
                pltpu.sync_copy(rows, acc.at[didx], add=True)
                return 0
            lax.fori_loop(0, nch, chunk, 0)

        @pl.when(c == 0)
        def _():
            run(hp_t, src_t, dst_t)

        @pl.when(c == 1)
        def _():
            run(hp_l, src_l, dst_l)

        plsc.subcore_barrier()
        for t in range(RT // RB):
            r0 = s * RT + t * RB
            pltpu.sync_copy(acc.at[pl.ds(r0, RB)], zb)
            pltpu.sync_copy(zb, out.at[c, pl.ds(r0, RB)])

    return pl.kernel(
        body,
        out_type=jax.ShapeDtypeStruct((NC, NP, W), jnp.float32),
        mesh=mesh,
        scratch_types=[
            pltpu.VMEM((CH,), jnp.int32),
            pltpu.VMEM((CH,), jnp.int32),
            pltpu.VMEM((CH, W), jnp.float32),
            pltpu.VMEM((RB, W), jnp.float32),
            pltpu.VMEM_SHARED((NP, W), jnp.float32),
            pltpu.SemaphoreType.DMA,
        ],
        compiler_params=pltpu.CompilerParams(use_tc_tiling_on_sc=False, needs_layout_passes=False),
    )


def _sortpool():
    NV = NN // L
    mesh = plsc.VectorSubcoreMesh(core_axis_name="c", subcore_axis_name="s")

    def body(y_t, lc_t, bt_t, y_l, lc_l, bt_l, out_t, out_l,
             btv, lcv, vals, posb, isel, rows, sem):
        c = lax.axis_index("c")
        s = lax.axis_index("s")
        lanes = lax.iota(jnp.int32, L)
        neg = jnp.full((L,), -jnp.inf, jnp.float32)

        def run(y, lc, bt, out):
            b = s
            pltpu.sync_copy(bt, btv)
            pltpu.sync_copy(lc.at[pl.ds(0, NN)], lcv)

            def pre(i, _):
                vals[pl.ds(i * L, L)] = neg
                posb[pl.ds(i * L, L)] = jnp.full((L,), DUMMY, jnp.int32)
                return 0
            lax.fori_loop(0, NP // L, pre, 0)
            isel[pl.ds(0, L)] = jnp.full((L,), DUMMY, jnp.int32)
            isel[pl.ds(L, L)] = jnp.full((L,), DUMMY, jnp.int32)

            def comp(k2, cnt):
                m = btv[pl.ds(k2 * L, L)] == b
                v = lcv[pl.ds(k2 * L, L)]
                pc = plsc.cumsum(m.astype(jnp.int32))
                idx = cnt + pc - 1
                plsc.store_scatter(vals, [idx], v, mask=m)
                plsc.store_scatter(posb, [idx], lanes + k2 * L, mask=m)
                return cnt + jnp.max(pc)
            cnt = lax.fori_loop(0, NV, comp, jnp.int32(0))
            nvec = (cnt + L - 1) // L

            def sel(t, _):
                mv = lax.fori_loop(
                    0, nvec,
                    lambda j, a: jnp.maximum(a, vals[pl.ds(j * L, L)]), neg)
                mx = jnp.max(mv)
                j = lax.while_loop(
                    lambda j: jnp.logical_not(
                        jnp.any(vals[pl.ds(j * L, L)] == mx)),
                    lambda j: j + 1, jnp.int32(0))
                v = vals[pl.ds(j * L, L)]
                eq = v == mx
                first = jnp.logical_and(eq, lanes == plsc.all_reduce_ffs(eq))
                pos = posb[pl.ds(j * L, L)]
                posx = jnp.where(mx == -jnp.inf,
                                 jnp.full((L,), DUMMY, jnp.int32), pos)
                plsc.store_scatter(isel, [jnp.full((L,), t, jnp.int32)],
                                   posx, mask=first)
                vals[pl.ds(j * L, L)] = jnp.where(first, neg, v)
                return 0
            lax.fori_loop(0, KP, sel, 0)

            pltpu.async_copy(y.at[isel], rows, sem).wait()
            pltpu.sync_copy(rows.at[pl.ds(0, KP)], out.at[b])

        @pl.when(c == 0)
        def _():
            run(y_t, lc_t, bt_t, out_t)

        @pl.when(c == 1)
        def _():
            run(y_l, lc_l, bt_l, out_l)

    return pl.kernel(
        body,
        out_type=(jax.ShapeDtypeStruct((NB, KP, 32), jnp.float32),
                  jax.ShapeDtypeStruct((NB, KP, 32), jnp.float32)),
        mesh=mesh,
        scratch_types=[
            pltpu.VMEM((NN,), jnp.int32),
            pltpu.VMEM((NN,), jnp.float32),
            pltpu.VMEM((NP,), jnp.float32),
            pltpu.VMEM((NP,), jnp.int32),
            pltpu.VMEM((2 * L,), jnp.int32),
            pltpu.VMEM((2 * L, 32), jnp.float32),
            pltpu.SemaphoreType.DMA,
        ],
        compiler_params=pltpu.CompilerParams(use_tc_tiling_on_sc=False, needs_layout_passes=False),
    )


def _dinv():
    def k(pref, oref):
        oref[...] = lax.rsqrt(pref[:, :, 0] + 1.0)
    return pl.pallas_call(
        k, grid=(NP // BR,),
        in_specs=[pl.BlockSpec((2, BR, 16), lambda i: (0, i, 0))],
        out_specs=pl.BlockSpec((2, BR), lambda i: (0, i)),
        out_shape=jax.ShapeDtypeStruct((2, NP), jnp.float32))


def _mm1():
    def k(xref, wref, dref, oref):
        h = jnp.dot(xref[...], wref[...], preferred_element_type=jnp.float32)
        oref[...] = dref[...][:, None] * h
    return pl.pallas_call(
        k, grid=(NP // BR,),
        in_specs=[pl.BlockSpec((BR, DF), lambda i: (i, 0)),
                  pl.BlockSpec((DF, 64), lambda i: (0, 0)),
                  pl.BlockSpec((BR,), lambda i: (i,))],
        out_specs=pl.BlockSpec((BR, 64), lambda i: (i, 0)),
        out_shape=jax.ShapeDtypeStruct((NP, 64), jnp.float32))


def _layer(b, W, Wo):
    def k(sref, href, dref, b1ref, lwref, lbref, wnref, oref):
        dv = dref[...][:, None]
        g = dv * (sref[0] + href[...]) + b1ref[...][None, :]
        x2 = jnp.where(g >= 0, g, 0.01 * g) + jnp.dot(
            g, lwref[...], preferred_element_type=jnp.float32) + lbref[...][None, :]
        oref[...] = dv * jnp.dot(x2, wnref[...],
                                 preferred_element_type=jnp.float32)
    return pl.pallas_call(
        k, grid=(NP // BR,),
        in_specs=[pl.BlockSpec((1, BR, W), lambda i: (b, i, 0)),
                  pl.BlockSpec((BR, W), lambda i: (i, 0)),
                  pl.BlockSpec((BR,), lambda i: (i,)),
                  pl.BlockSpec((W,), lambda i: (0,)),
                  pl.BlockSpec((W, W), lambda i: (0, 0)),
                  pl.BlockSpec((W,), lambda i: (0,)),
                  pl.BlockSpec((W, Wo), lambda i: (0, 0))],
        out_specs=pl.BlockSpec((BR, Wo), lambda i: (i, 0)),
        out_shape=jax.ShapeDtypeStruct((NP, Wo), jnp.float32))


def _final(b):
    def k(sref, href, dref, boref, yref, lref):
        dv = dref[...][:, None]
        g = dv * (sref[0] + href[...]) + boref[...][None, :]
        rid = lax.broadcasted_iota(jnp.int32, (BR, 1), 0) + pl.program_id(0) * BR
        y = jnp.where(rid < NN, g, 0.0)
        yref[...] = y
        lref[...] = y[:, 31]
    return pl.pallas_call(
        k, grid=(NP // BR,),
        in_specs=[pl.BlockSpec((1, BR, 32), lambda i: (b, i, 0)),
                  pl.BlockSpec((BR, 32), lambda i: (i, 0)),
                  pl.BlockSpec((BR,), lambda i: (i,)),
                  pl.BlockSpec((32,), lambda i: (0,))],
        out_specs=[pl.BlockSpec((BR, 32), lambda i: (i, 0)),
                   pl.BlockSpec((BR,), lambda i: (i,))],
        out_shape=[jax.ShapeDtypeStruct((NP, 32), jnp.float32),
                   jax.ShapeDtypeStruct((NP,), jnp.float32)])


def _mlp():
    def k(xt, xl, w1, b1, w2, b2, wo, bo, oref):
        w1f = w1[...]
        a = (jnp.dot(xt[...], w1f[:KP * 32], preferred_element_type=jnp.float32)
             + jnp.dot(xl[...], w1f[KP * 32:], preferred_element_type=jnp.float32)
             + b1[...][None, :])
        a = jnp.where(a >= 0, a, 0.01 * a)
        h = jnp.dot(a, w2[...], preferred_element_type=jnp.float32) + b2[...][None, :]
        h = jnp.where(h >= 0, h, 0.01 * h)
        oref[...] = jnp.dot(h, wo[...], preferred_element_type=jnp.float32) + bo[...][None, :]
    return pl.pallas_call(
        k, out_shape=jax.ShapeDtypeStruct((NB, 1), jnp.float32))


_scat16 = _sc_scatter(16)
_scat64 = _sc_scatter(64)
_scat32 = _sc_scatter(32)
_sortp = _sortpool()
_dinv_k = _dinv()
_mm1_k = _mm1()
_lay1t = _layer(0, 64, 64)
_lay1l = _layer(1, 64, 64)
_lay2t = _layer(0, 64, 32)
_lay2l = _layer(1, 64, 32)
_fin_t = _final(0)
_fin_l = _final(1)
_mlp_k = _mlp()


def _pad_edges(e):
    return jnp.pad(e, (0, EP - NE), constant_values=DUMMY)


def kernel(x_topo, edge_index_topo, x_topo_batch, x_lc, edge_index_lc,
           x_lc_batch, topo_params, lc_params, mlp_params):
    f32 = jnp.float32
    xt = jnp.pad(x_topo.astype(f32), ((0, NP - NN), (0, 0)))
    xl = jnp.pad(x_lc.astype(f32), ((0, NP - NN), (0, 0)))
    src_t = _pad_edges(edge_index_topo[0])
    dst_t = _pad_edges(edge_index_topo[1])
    src_l = _pad_edges(edge_index_lc[0])
    dst_l = _pad_edges(edge_index_lc[1])
    ones = jnp.ones((NP, 16), f32)
    tp, lp = topo_params, lc_params

    degp = _scat16(ones, dst_t, dst_t, ones, dst_l, dst_l)
    dinv2 = _dinv_k(degp)
    dvt, dvl = dinv2[0], dinv2[1]

    hp1t = _mm1_k(xt, tp[0], dvt)
    hp1l = _mm1_k(xl, lp[0], dvl)
    S1 = _scat64(hp1t, src_t, dst_t, hp1l, src_l, dst_l)

    hp2t = _lay1t(S1, hp1t, dvt, tp[1], tp[2], tp[3], tp[4])
    hp2l = _lay1l(S1, hp1l, dvl, lp[1], lp[2], lp[3], lp[4])
    S2 = _scat64(hp2t, src_t, dst_t, hp2l, src_l, dst_l)

    hp3t = _lay2t(S2, hp2t, dvt, tp[5], tp[6], tp[7], tp[8])
    hp3l = _lay2l(S2, hp2l, dvl, lp[5], lp[6], lp[7], lp[8])
    S3 = _scat32(hp3t, src_t, dst_t, hp3l, src_l, dst_l)

    yt, lct = _fin_t(S3, hp3t, dvt, tp[9])
    yl, lcl = _fin_l(S3, hp3l, dvl, lp[9])

    pt, plc = _sortp(yt, lct, x_topo_batch, yl, lcl, x_lc_batch)
    mW1, mb1, mW2, mb2, mWo, mbo = mlp_params
    return _mlp_k(pt.reshape(NB, KP * 32), plc.reshape(NB, KP * 32),
                  mW1, mb1, mW2, mb2, mWo, mbo)


# speedup vs baseline: 17.6387x; 1.8555x over previous
import functools

import jax
import jax.numpy as jnp
from jax import lax
from jax.experimental import pallas as pl
from jax.experimental.pallas import tpu as pltpu
from jax.experimental.pallas import tpu_sc as plsc

NB = 16
KP = 30
NN = 10000
NE = 320000
DF = 128
NP = 10240
DUMMY = 10200
NC, NS, L = 2, 16, 16
CH = 128
EPT_P = 20480
EP = EPT_P * NS
BR = 1024


NCH = EPT_P // CH
NBUF = 2


def _zero_acc(s, zb, acc, W):
    RT = NP // NS

    def zloop(i, _):
        r = i // (W // L)
        q = i % (W // L)
        zb[r, pl.ds(q * L, L)] = jnp.zeros((L,), jnp.float32)
        return 0
    lax.fori_loop(0, CH * (W // L), zloop, 0)
    for t in range(RT // CH):
        pltpu.sync_copy(zb, acc.at[pl.ds(s * RT + t * CH, CH)])


def _read_out(c, s, zb, acc, out):
    RT = NP // NS
    for t in range(RT // CH):
        r0 = s * RT + t * CH
        pltpu.sync_copy(acc.at[pl.ds(r0, CH)], zb)
        pltpu.sync_copy(zb, out.at[c, pl.ds(r0, CH)])


def _sc_scatter(W):
    mesh = plsc.VectorSubcoreMesh(core_axis_name="c", subcore_axis_name="s")

    def body(hp_t, s2_t, d2_t, hp_l, s2_l, d2_l, out,
             sidx2, didx2, r0b, r1b, r2b, r3b,
             zb, acc, semg, sems):
        c = lax.axis_index("c")
        s = lax.axis_index("s")
        rows = [r0b, r1b, r2b, r3b]
        grpA, grpB = rows[:NBUF], rows[NBUF:]

        _zero_acc(s, zb, acc, W)
        plsc.subcore_barrier()

        def run(hp, s2, d2):
            pltpu.sync_copy(s2.at[pl.ds(s * NCH, NCH)], sidx2)
            pltpu.sync_copy(d2.at[pl.ds(s * NCH, NCH)], didx2)
            for b in range(NBUF):
                pltpu.async_copy(hp.at[sidx2.at[b]], grpA[b], semg)

            def phase(j0, cur, nxt):
                for b in range(NBUF):
                    pltpu.make_async_copy(hp.at[sidx2.at[j0 + b]],
                                          cur[b], semg).wait()

                @pl.when(j0 + NBUF < NCH)
                def _():
                    for b in range(NBUF):
                        pltpu.async_copy(hp.at[sidx2.at[j0 + NBUF + b]],
                                         nxt[b], semg)
                sc = [pltpu.async_copy(cur[b], acc.at[didx2.at[j0 + b]],
                                       sems, add=True) for b in range(NBUF)]
                for d in sc:
                    d.wait()

            def sup(i, _):
                j0 = i * NBUF

                @pl.when(i % 2 == 0)
                def _():
                    phase(j0, grpA, grpB)

                @pl.when(i % 2 == 1)
                def _():
                    phase(j0, grpB, grpA)
                return 0
            lax.fori_loop(0, NCH // NBUF, sup, 0)

        @pl.when(c == 0)
        def _():
            run(hp_t, s2_t, d2_t)

        @pl.when(c == 1)
        def _():
            run(hp_l, s2_l, d2_l)

        plsc.subcore_barrier()
        _read_out(c, s, zb, acc, out)

    return pl.kernel(
        body,
        out_type=jax.ShapeDtypeStruct((NC, NP, W), jnp.float32),
        mesh=mesh,
        scratch_types=[
            pltpu.VMEM((NCH, CH), jnp.int32),
            pltpu.VMEM((NCH, CH), jnp.int32),
        ] + [pltpu.VMEM((CH, W), jnp.float32)] * (2 * NBUF) + [
            pltpu.VMEM((CH, W), jnp.float32),
            pltpu.VMEM_SHARED((NP, W), jnp.float32),
            pltpu.SemaphoreType.DMA,
            pltpu.SemaphoreType.DMA,
        ],
        compiler_params=pltpu.CompilerParams(use_tc_tiling_on_sc=False, needs_layout_passes=False),
    )


def _sc_deg():
    W = 16
    KF = 8
    mesh = plsc.VectorSubcoreMesh(core_axis_name="c", subcore_axis_name="s")

    def body(d2_t, d2_l, out, didx2, ones, zb, acc, sems):
        c = lax.axis_index("c")
        s = lax.axis_index("s")

        def oloop(i, _):
            ones[i, pl.ds(0, L)] = jnp.full((L,), 1.0, jnp.float32)
            return 0
        lax.fori_loop(0, CH, oloop, 0)
        _zero_acc(s, zb, acc, W)
        plsc.subcore_barrier()

        def run(d2):
            pltpu.sync_copy(d2.at[pl.ds(s * NCH, NCH)], didx2)

            def sup(i, _):
                sc = [pltpu.async_copy(ones, acc.at[didx2.at[i * KF + b]],
                                       sems, add=True) for b in range(KF)]
                for d in sc:
                    d.wait()
                return 0
            lax.fori_loop(0, NCH // KF, sup, 0)

        @pl.when(c == 0)
        def _():
            run(d2_t)

        @pl.when(c == 1)
        def _():
            run(d2_l)

        plsc.subcore_barrier()
        _read_out(c, s, zb, acc, out)

    return pl.kernel(
        body,
        out_type=jax.ShapeDtypeStruct((NC, NP, W), jnp.float32),
        mesh=mesh,
        scratch_types=[
            pltpu.VMEM((NCH, CH), jnp.int32),
            pltpu.VMEM((CH, W), jnp.float32),
            pltpu.VMEM((CH, W), jnp.float32),
            pltpu.VMEM_SHARED((NP, W), jnp.float32),
            pltpu.SemaphoreType.DMA,
        ],
        compiler_params=pltpu.CompilerParams(use_tc_tiling_on_sc=False, needs_layout_passes=False),
    )


def _sortpool():
    NV = NN // L
    mesh = plsc.VectorSubcoreMesh(core_axis_name="c", subcore_axis_name="s")

    def body(y_t, lc_t, bt_t, y_l, lc_l, bt_l, out_t, out_l,
             btv, lcv, vals, posb, isel, rows, sem):
        c = lax.axis_index("c")
        s = lax.axis_index("s")
        lanes = lax.iota(jnp.int32, L)
        neg = jnp.full((L,), -jnp.inf, jnp.float32)

        def run(y, lc, bt, out):
            b = s
            pltpu.sync_copy(bt, btv)
            pltpu.sync_copy(lc.at[pl.ds(0, NN)], lcv)

            def pre(i, _):
                vals[pl.ds(i * L, L)] = neg
                posb[pl.ds(i * L, L)] = jnp.full((L,), DUMMY, jnp.int32)
                return 0
            lax.fori_loop(0, NP // L, pre, 0)
            isel[pl.ds(0, L)] = jnp.full((L,), DUMMY, jnp.int32)
            isel[pl.ds(L, L)] = jnp.full((L,), DUMMY, jnp.int32)

            def comp(k2, cnt):
                m = btv[pl.ds(k2 * L, L)] == b
                v = lcv[pl.ds(k2 * L, L)]
                pc = plsc.cumsum(m.astype(jnp.int32))
                idx = cnt + pc - 1
                plsc.store_scatter(vals, [idx], v, mask=m)
                plsc.store_scatter(posb, [idx], lanes + k2 * L, mask=m)
                return cnt + jnp.max(pc)
            cnt = lax.fori_loop(0, NV, comp, jnp.int32(0))
            nvec = (cnt + L - 1) // L

            def sel(t, _):
                mv = lax.fori_loop(
                    0, nvec,
                    lambda j, a: jnp.maximum(a, vals[pl.ds(j * L, L)]), neg)
                mx = jnp.max(mv)
                j = lax.while_loop(
                    lambda j: jnp.logical_not(
                        jnp.any(vals[pl.ds(j * L, L)] == mx)),
                    lambda j: j + 1, jnp.int32(0))
                v = vals[pl.ds(j * L, L)]
                eq = v == mx
                first = jnp.logical_and(eq, lanes == plsc.all_reduce_ffs(eq))
                pos = posb[pl.ds(j * L, L)]
                posx = jnp.where(mx == -jnp.inf,
                                 jnp.full((L,), DUMMY, jnp.int32), pos)
                plsc.store_scatter(isel, [jnp.full((L,), t, jnp.int32)],
                                   posx, mask=first)
                vals[pl.ds(j * L, L)] = jnp.where(first, neg, v)
                return 0
            lax.fori_loop(0, KP, sel, 0)

            pltpu.async_copy(y.at[isel], rows, sem).wait()
            pltpu.sync_copy(rows.at[pl.ds(0, KP)], out.at[b])

        @pl.when(c == 0)
        def _():
            run(y_t, lc_t, bt_t, out_t)

        @pl.when(c == 1)
        def _():
            run(y_l, lc_l, bt_l, out_l)

    return pl.kernel(
        body,
        out_type=(jax.ShapeDtypeStruct((NB, KP, 32), jnp.float32),
                  jax.ShapeDtypeStruct((NB, KP, 32), jnp.float32)),
        mesh=mesh,
        scratch_types=[
            pltpu.VMEM((NN,), jnp.int32),
            pltpu.VMEM((NN,), jnp.float32),
            pltpu.VMEM((NP,), jnp.float32),
            pltpu.VMEM((NP,), jnp.int32),
            pltpu.VMEM((2 * L,), jnp.int32),
            pltpu.VMEM((2 * L, 32), jnp.float32),
            pltpu.SemaphoreType.DMA,
        ],
        compiler_params=pltpu.CompilerParams(use_tc_tiling_on_sc=False, needs_layout_passes=False),
    )


def _dinv():
    def k(pref, oref):
        oref[...] = lax.rsqrt(pref[:, :, 0] + 1.0)
    return pl.pallas_call(
        k, grid=(NP // BR,),
        in_specs=[pl.BlockSpec((2, BR, 16), lambda i: (0, i, 0))],
        out_specs=pl.BlockSpec((2, BR), lambda i: (0, i)),
        out_shape=jax.ShapeDtypeStruct((2, NP), jnp.float32))


def _mm1():
    def k(xref, wref, dref, oref):
        h = jnp.dot(xref[...], wref[...], preferred_element_type=jnp.float32)
        oref[...] = dref[...][:, None] * h
    return pl.pallas_call(
        k, grid=(NP // BR,),
        in_specs=[pl.BlockSpec((BR, DF), lambda i: (i, 0)),
                  pl.BlockSpec((DF, 64), lambda i: (0, 0)),
                  pl.BlockSpec((BR,), lambda i: (i,))],
        out_specs=pl.BlockSpec((BR, 64), lambda i: (i, 0)),
        out_shape=jax.ShapeDtypeStruct((NP, 64), jnp.float32))


def _layer(b, W, Wo):
    def k(sref, href, dref, b1ref, lwref, lbref, wnref, oref):
        dv = dref[...][:, None]
        g = dv * (sref[0] + href[...]) + b1ref[...][None, :]
        x2 = jnp.where(g >= 0, g, 0.01 * g) + jnp.dot(
            g, lwref[...], preferred_element_type=jnp.float32) + lbref[...][None, :]
        oref[...] = dv * jnp.dot(x2, wnref[...],
                                 preferred_element_type=jnp.float32)
    return pl.pallas_call(
        k, grid=(NP // BR,),
        in_specs=[pl.BlockSpec((1, BR, W), lambda i: (b, i, 0)),
                  pl.BlockSpec((BR, W), lambda i: (i, 0)),
                  pl.BlockSpec((BR,), lambda i: (i,)),
                  pl.BlockSpec((W,), lambda i: (0,)),
                  pl.BlockSpec((W, W), lambda i: (0, 0)),
                  pl.BlockSpec((W,), lambda i: (0,)),
                  pl.BlockSpec((W, Wo), lambda i: (0, 0))],
        out_specs=pl.BlockSpec((BR, Wo), lambda i: (i, 0)),
        out_shape=jax.ShapeDtypeStruct((NP, Wo), jnp.float32))


def _final(b):
    def k(sref, href, dref, boref, yref, lref):
        dv = dref[...][:, None]
        g = dv * (sref[0] + href[...]) + boref[...][None, :]
        rid = lax.broadcasted_iota(jnp.int32, (BR, 1), 0) + pl.program_id(0) * BR
        y = jnp.where(rid < NN, g, 0.0)
        yref[...] = y
        lref[...] = y[:, 31]
    return pl.pallas_call(
        k, grid=(NP // BR,),
        in_specs=[pl.BlockSpec((1, BR, 32), lambda i: (b, i, 0)),
                  pl.BlockSpec((BR, 32), lambda i: (i, 0)),
                  pl.BlockSpec((BR,), lambda i: (i,)),
                  pl.BlockSpec((32,), lambda i: (0,))],
        out_specs=[pl.BlockSpec((BR, 32), lambda i: (i, 0)),
                   pl.BlockSpec((BR,), lambda i: (i,))],
        out_shape=[jax.ShapeDtypeStruct((NP, 32), jnp.float32),
                   jax.ShapeDtypeStruct((NP,), jnp.float32)])


def _mlp():
    def k(xt, xl, w1, b1, w2, b2, wo, bo, oref):
        w1f = w1[...]
        a = (jnp.dot(xt[...], w1f[:KP * 32], preferred_element_type=jnp.float32)
             + jnp.dot(xl[...], w1f[KP * 32:], preferred_element_type=jnp.float32)
             + b1[...][None, :])
        a = jnp.where(a >= 0, a, 0.01 * a)
        h = jnp.dot(a, w2[...], preferred_element_type=jnp.float32) + b2[...][None, :]
        h = jnp.where(h >= 0, h, 0.01 * h)
        oref[...] = jnp.dot(h, wo[...], preferred_element_type=jnp.float32) + bo[...][None, :]
    return pl.pallas_call(
        k, out_shape=jax.ShapeDtypeStruct((NB, 1), jnp.float32))


_deg_k = _sc_deg()
_scat64 = _sc_scatter(64)
_scat32 = _sc_scatter(32)
_sortp = _sortpool()
_dinv_k = _dinv()
_mm1_k = _mm1()
_lay1t = _layer(0, 64, 64)
_lay1l = _layer(1, 64, 64)
_lay2t = _layer(0, 64, 32)
_lay2l = _layer(1, 64, 32)
_fin_t = _final(0)
_fin_l = _final(1)
_mlp_k = _mlp()


def _pad_edges(e):
    return jnp.pad(e, (0, EP - NE),
                   constant_values=DUMMY).reshape(NS * NCH, CH)


def kernel(x_topo, edge_index_topo, x_topo_batch, x_lc, edge_index_lc,
           x_lc_batch, topo_params, lc_params, mlp_params):
    f32 = jnp.float32
    xt = jnp.pad(x_topo.astype(f32), ((0, NP - NN), (0, 0)))
    xl = jnp.pad(x_lc.astype(f32), ((0, NP - NN), (0, 0)))
    src_t = _pad_edges(edge_index_topo[0])
    dst_t = _pad_edges(edge_index_topo[1])
    src_l = _pad_edges(edge_index_lc[0])
    dst_l = _pad_edges(edge_index_lc[1])
    tp, lp = topo_params, lc_params

    degp = _deg_k(dst_t, dst_l)
    dinv2 = _dinv_k(degp)
    dvt, dvl = dinv2[0], dinv2[1]

    hp1t = _mm1_k(xt, tp[0], dvt)
    hp1l = _mm1_k(xl, lp[0], dvl)
    S1 = _scat64(hp1t, src_t, dst_t, hp1l, src_l, dst_l)

    hp2t = _lay1t(S1, hp1t, dvt, tp[1], tp[2], tp[3], tp[4])
    hp2l = _lay1l(S1, hp1l, dvl, lp[1], lp[2], lp[3], lp[4])
    S2 = _scat64(hp2t, src_t, dst_t, hp2l, src_l, dst_l)

    hp3t = _lay2t(S2, hp2t, dvt, tp[5], tp[6], tp[7], tp[8])
    hp3l = _lay2l(S2, hp2l, dvl, lp[5], lp[6], lp[7], lp[8])
    S3 = _scat32(hp3t, src_t, dst_t, hp3l, src_l, dst_l)

    yt, lct = _fin_t(S3, hp3t, dvt, tp[9])
    yl, lcl = _fin_l(S3, hp3l, dvl, lp[9])

    pt, plc = _sortp(yt, lct, x_topo_batch, yl, lcl, x_lc_batch)
    mW1, mb1, mW2, mb2, mWo, mbo = mlp_params
    return _mlp_k(pt.reshape(NB, KP * 32), plc.reshape(NB, KP * 32),
                  mW1, mb1, mW2, mb2, mWo, mbo)


# speedup vs baseline: 18.0766x; 1.0248x over previous
import functools

import jax
import jax.numpy as jnp
from jax import lax
from jax.experimental import pallas as pl
from jax.experimental.pallas import tpu as pltpu
from jax.experimental.pallas import tpu_sc as plsc

NB = 16
KP = 30
NN = 10000
NE = 320000
DF = 128
NP = 10240
DUMMY = 10200
NC, NS, L = 2, 16, 16
CH = 128
EPT_P = 20480
EP = EPT_P * NS
BR = 1024


NCH = EPT_P // CH
NBUF = 4
SB = 32
NBLK = NCH // SB


def _zero_acc(s, zb, acc, W):
    RT = NP // NS

    def zloop(i, _):
        r = i // (W // L)
        q = i % (W // L)
        zb[r, pl.ds(q * L, L)] = jnp.zeros((L,), jnp.float32)
        return 0
    lax.fori_loop(0, CH * (W // L), zloop, 0)
    for t in range(RT // CH):
        pltpu.sync_copy(zb, acc.at[pl.ds(s * RT + t * CH, CH)])


def _read_out(c, s, zb, acc, out):
    RT = NP // NS
    for t in range(RT // CH):
        r0 = s * RT + t * CH
        pltpu.sync_copy(acc.at[pl.ds(r0, CH)], zb)
        pltpu.sync_copy(zb, out.at[c, pl.ds(r0, CH)])


def _sc_scatter(W):
    mesh = plsc.VectorSubcoreMesh(core_axis_name="c", subcore_axis_name="s")

    def body(hp_t, s2_t, d2_t, hp_l, s2_l, d2_l, out,
             sa_s, sa_d, sb_s, sb_d, r0b, r1b, r2b, r3b, r4b, r5b, r6b, r7b,
             zb, acc, semg, sems, semi):
        c = lax.axis_index("c")
        s = lax.axis_index("s")
        rows = [r0b, r1b, r2b, r3b, r4b, r5b, r6b, r7b]
        grpA, grpB = rows[:NBUF], rows[NBUF:]

        _zero_acc(s, zb, acc, W)
        plsc.subcore_barrier()

        def run(hp, s2, d2):
            pltpu.async_copy(s2.at[pl.ds(s * NCH, SB)], sa_s, semi)
            pltpu.async_copy(d2.at[pl.ds(s * NCH, SB)], sa_d, semi)
            for blk in range(NBLK):
                si, di = (sa_s, sa_d) if blk % 2 == 0 else (sb_s, sb_d)
                sn, dn = (sb_s, sb_d) if blk % 2 == 0 else (sa_s, sa_d)
                pltpu.make_async_copy(s2.at[pl.ds(0, SB)], si, semi).wait()
                pltpu.make_async_copy(d2.at[pl.ds(0, SB)], di, semi).wait()
                if blk + 1 < NBLK:
                    nb = s * NCH + (blk + 1) * SB
                    pltpu.async_copy(s2.at[pl.ds(nb, SB)], sn, semi)
                    pltpu.async_copy(d2.at[pl.ds(nb, SB)], dn, semi)
                for b in range(NBUF):
                    pltpu.async_copy(hp.at[si.at[b]], grpA[b], semg)

                def phase(j0, cur, nxt):
                    for b in range(NBUF):
                        pltpu.make_async_copy(hp.at[si.at[j0 + b]],
                                              cur[b], semg).wait()

                    @pl.when(j0 + NBUF < SB)
                    def _():
                        for b in range(NBUF):
                            pltpu.async_copy(hp.at[si.at[j0 + NBUF + b]],
                                             nxt[b], semg)
                    sc = [pltpu.async_copy(cur[b], acc.at[di.at[j0 + b]],
                                           sems, add=True)
                          for b in range(NBUF)]
                    for d in sc:
                        d.wait()

                def sup(i, _):
                    j0 = i * NBUF

                    @pl.when(i % 2 == 0)
                    def _():
                        phase(j0, grpA, grpB)

                    @pl.when(i % 2 == 1)
                    def _():
                        phase(j0, grpB, grpA)
                    return 0
                lax.fori_loop(0, SB // NBUF, sup, 0)

        @pl.when(c == 0)
        def _():
            run(hp_t, s2_t, d2_t)

        @pl.when(c == 1)
        def _():
            run(hp_l, s2_l, d2_l)

        plsc.subcore_barrier()
        _read_out(c, s, zb, acc, out)

    return pl.kernel(
        body,
        out_type=jax.ShapeDtypeStruct((NC, NP, W), jnp.float32),
        mesh=mesh,
        scratch_types=[
            pltpu.VMEM((SB, CH), jnp.int32),
            pltpu.VMEM((SB, CH), jnp.int32),
            pltpu.VMEM((SB, CH), jnp.int32),
            pltpu.VMEM((SB, CH), jnp.int32),
        ] + [pltpu.VMEM((CH, W), jnp.float32)] * (2 * NBUF) + [
            pltpu.VMEM((CH, W), jnp.float32),
            pltpu.VMEM_SHARED((NP, W), jnp.float32),
            pltpu.SemaphoreType.DMA,
            pltpu.SemaphoreType.DMA,
            pltpu.SemaphoreType.DMA,
        ],
        compiler_params=pltpu.CompilerParams(use_tc_tiling_on_sc=False, needs_layout_passes=False),
    )


def _sc_deg():
    W = 16
    KF = 8
    mesh = plsc.VectorSubcoreMesh(core_axis_name="c", subcore_axis_name="s")

    def body(d2_t, d2_l, out, didx2, ones, zb, acc, sems):
        c = lax.axis_index("c")
        s = lax.axis_index("s")

        def oloop(i, _):
            ones[i, pl.ds(0, L)] = jnp.full((L,), 1.0, jnp.float32)
            return 0
        lax.fori_loop(0, CH, oloop, 0)
        _zero_acc(s, zb, acc, W)
        plsc.subcore_barrier()

        def run(d2):
            pltpu.sync_copy(d2.at[pl.ds(s * NCH, NCH)], didx2)

            def sup(i, _):
                sc = [pltpu.async_copy(ones, acc.at[didx2.at[i * KF + b]],
                                       sems, add=True) for b in range(KF)]
                for d in sc:
                    d.wait()
                return 0
            lax.fori_loop(0, NCH // KF, sup, 0)

        @pl.when(c == 0)
        def _():
            run(d2_t)

        @pl.when(c == 1)
        def _():
            run(d2_l)

        plsc.subcore_barrier()
        _read_out(c, s, zb, acc, out)

    return pl.kernel(
        body,
        out_type=jax.ShapeDtypeStruct((NC, NP, W), jnp.float32),
        mesh=mesh,
        scratch_types=[
            pltpu.VMEM((NCH, CH), jnp.int32),
            pltpu.VMEM((CH, W), jnp.float32),
            pltpu.VMEM((CH, W), jnp.float32),
            pltpu.VMEM_SHARED((NP, W), jnp.float32),
            pltpu.SemaphoreType.DMA,
        ],
        compiler_params=pltpu.CompilerParams(use_tc_tiling_on_sc=False, needs_layout_passes=False),
    )


def _sortpool():
    NV = NN // L
    mesh = plsc.VectorSubcoreMesh(core_axis_name="c", subcore_axis_name="s")

    def body(y_t, lc_t, bt_t, y_l, lc_l, bt_l, out_t, out_l,
             btv, lcv, vals, posb, isel, rows, sem):
        c = lax.axis_index("c")
        s = lax.axis_index("s")
        lanes = lax.iota(jnp.int32, L)
        neg = jnp.full((L,), -jnp.inf, jnp.float32)

        def run(y, lc, bt, out):
            b = s
            pltpu.sync_copy(bt, btv)
            pltpu.sync_copy(lc.at[pl.ds(0, NN)], lcv)

            def pre(i, _):
                vals[pl.ds(i * L, L)] = neg
                posb[pl.ds(i * L, L)] = jnp.full((L,), DUMMY, jnp.int32)
                return 0
            lax.fori_loop(0, NP // L, pre, 0)
            isel[pl.ds(0, L)] = jnp.full((L,), DUMMY, jnp.int32)
            isel[pl.ds(L, L)] = jnp.full((L,), DUMMY, jnp.int32)

            def comp(k2, cnt):
                m = btv[pl.ds(k2 * L, L)] == b
                v = lcv[pl.ds(k2 * L, L)]
                pc = plsc.cumsum(m.astype(jnp.int32))
                idx = cnt + pc - 1
                plsc.store_scatter(vals, [idx], v, mask=m)
                plsc.store_scatter(posb, [idx], lanes + k2 * L, mask=m)
                return cnt + jnp.max(pc)
            cnt = lax.fori_loop(0, NV, comp, jnp.int32(0))
            nvec = (cnt + L - 1) // L

            def sel(t, _):
                mv = lax.fori_loop(
                    0, nvec,
                    lambda j, a: jnp.maximum(a, vals[pl.ds(j * L, L)]), neg)
                mx = jnp.max(mv)
                j = lax.while_loop(
                    lambda j: jnp.logical_not(
                        jnp.any(vals[pl.ds(j * L, L)] == mx)),
                    lambda j: j + 1, jnp.int32(0))
                v = vals[pl.ds(j * L, L)]
                eq = v == mx
                first = jnp.logical_and(eq, lanes == plsc.all_reduce_ffs(eq))
                pos = posb[pl.ds(j * L, L)]
                posx = jnp.where(mx == -jnp.inf,
                                 jnp.full((L,), DUMMY, jnp.int32), pos)
                plsc.store_scatter(isel, [jnp.full((L,), t, jnp.int32)],
                                   posx, mask=first)
                vals[pl.ds(j * L, L)] = jnp.where(first, neg, v)
                return 0
            lax.fori_loop(0, KP, sel, 0)

            pltpu.async_copy(y.at[isel], rows, sem).wait()
            pltpu.sync_copy(rows.at[pl.ds(0, KP)], out.at[b])

        @pl.when(c == 0)
        def _():
            run(y_t, lc_t, bt_t, out_t)

        @pl.when(c == 1)
        def _():
            run(y_l, lc_l, bt_l, out_l)

    return pl.kernel(
        body,
        out_type=(jax.ShapeDtypeStruct((NB, KP, 32), jnp.float32),
                  jax.ShapeDtypeStruct((NB, KP, 32), jnp.float32)),
        mesh=mesh,
        scratch_types=[
            pltpu.VMEM((NN,), jnp.int32),
            pltpu.VMEM((NN,), jnp.float32),
            pltpu.VMEM((NP,), jnp.float32),
            pltpu.VMEM((NP,), jnp.int32),
            pltpu.VMEM((2 * L,), jnp.int32),
            pltpu.VMEM((2 * L, 32), jnp.float32),
            pltpu.SemaphoreType.DMA,
        ],
        compiler_params=pltpu.CompilerParams(use_tc_tiling_on_sc=False, needs_layout_passes=False),
    )


def _dinv():
    def k(pref, oref):
        oref[...] = lax.rsqrt(pref[:, :, 0] + 1.0)
    return pl.pallas_call(
        k, grid=(NP // BR,),
        in_specs=[pl.BlockSpec((2, BR, 16), lambda i: (0, i, 0))],
        out_specs=pl.BlockSpec((2, BR), lambda i: (0, i)),
        out_shape=jax.ShapeDtypeStruct((2, NP), jnp.float32))


def _mm1():
    def k(xref, wref, dref, oref):
        h = jnp.dot(xref[...], wref[...], preferred_element_type=jnp.float32)
        oref[...] = dref[...][:, None] * h
    return pl.pallas_call(
        k, grid=(NP // BR,),
        in_specs=[pl.BlockSpec((BR, DF), lambda i: (i, 0)),
                  pl.BlockSpec((DF, 64), lambda i: (0, 0)),
                  pl.BlockSpec((BR,), lambda i: (i,))],
        out_specs=pl.BlockSpec((BR, 64), lambda i: (i, 0)),
        out_shape=jax.ShapeDtypeStruct((NP, 64), jnp.float32))


def _layer(b, W, Wo):
    def k(sref, href, dref, b1ref, lwref, lbref, wnref, oref):
        dv = dref[...][:, None]
        g = dv * (sref[0] + href[...]) + b1ref[...][None, :]
        x2 = jnp.where(g >= 0, g, 0.01 * g) + jnp.dot(
            g, lwref[...], preferred_element_type=jnp.float32) + lbref[...][None, :]
        oref[...] = dv * jnp.dot(x2, wnref[...],
                                 preferred_element_type=jnp.float32)
    return pl.pallas_call(
        k, grid=(NP // BR,),
        in_specs=[pl.BlockSpec((1, BR, W), lambda i: (b, i, 0)),
                  pl.BlockSpec((BR, W), lambda i: (i, 0)),
                  pl.BlockSpec((BR,), lambda i: (i,)),
                  pl.BlockSpec((W,), lambda i: (0,)),
                  pl.BlockSpec((W, W), lambda i: (0, 0)),
                  pl.BlockSpec((W,), lambda i: (0,)),
                  pl.BlockSpec((W, Wo), lambda i: (0, 0))],
        out_specs=pl.BlockSpec((BR, Wo), lambda i: (i, 0)),
        out_shape=jax.ShapeDtypeStruct((NP, Wo), jnp.float32))


def _final(b):
    def k(sref, href, dref, boref, yref, lref):
        dv = dref[...][:, None]
        g = dv * (sref[0] + href[...]) + boref[...][None, :]
        rid = lax.broadcasted_iota(jnp.int32, (BR, 1), 0) + pl.program_id(0) * BR
        y = jnp.where(rid < NN, g, 0.0)
        yref[...] = y
        lref[...] = y[:, 31]
    return pl.pallas_call(
        k, grid=(NP // BR,),
        in_specs=[pl.BlockSpec((1, BR, 32), lambda i: (b, i, 0)),
                  pl.BlockSpec((BR, 32), lambda i: (i, 0)),
                  pl.BlockSpec((BR,), lambda i: (i,)),
                  pl.BlockSpec((32,), lambda i: (0,))],
        out_specs=[pl.BlockSpec((BR, 32), lambda i: (i, 0)),
                   pl.BlockSpec((BR,), lambda i: (i,))],
        out_shape=[jax.ShapeDtypeStruct((NP, 32), jnp.float32),
                   jax.ShapeDtypeStruct((NP,), jnp.float32)])


def _mlp():
    def k(xt, xl, w1, b1, w2, b2, wo, bo, oref):
        w1f = w1[...]
        a = (jnp.dot(xt[...], w1f[:KP * 32], preferred_element_type=jnp.float32)
             + jnp.dot(xl[...], w1f[KP * 32:], preferred_element_type=jnp.float32)
             + b1[...][None, :])
        a = jnp.where(a >= 0, a, 0.01 * a)
        h = jnp.dot(a, w2[...], preferred_element_type=jnp.float32) + b2[...][None, :]
        h = jnp.where(h >= 0, h, 0.01 * h)
        oref[...] = jnp.dot(h, wo[...], preferred_element_type=jnp.float32) + bo[...][None, :]
    return pl.pallas_call(
        k, out_shape=jax.ShapeDtypeStruct((NB, 1), jnp.float32))


_deg_k = _sc_deg()
_scat64 = _sc_scatter(64)
_scat32 = _sc_scatter(32)
_sortp = _sortpool()
_dinv_k = _dinv()
_mm1_k = _mm1()
_lay1t = _layer(0, 64, 64)
_lay1l = _layer(1, 64, 64)
_lay2t = _layer(0, 64, 32)
_lay2l = _layer(1, 64, 32)
_fin_t = _final(0)
_fin_l = _final(1)
_mlp_k = _mlp()


def _pad_edges(e):
    return jnp.pad(e, (0, EP - NE),
                   constant_values=DUMMY).reshape(NS * NCH, CH)


def kernel(x_topo, edge_index_topo, x_topo_batch, x_lc, edge_index_lc,
           x_lc_batch, topo_params, lc_params, mlp_params):
    f32 = jnp.float32
    xt = jnp.pad(x_topo.astype(f32), ((0, NP - NN), (0, 0)))
    xl = jnp.pad(x_lc.astype(f32), ((0, NP - NN), (0, 0)))
    src_t = _pad_edges(edge_index_topo[0])
    dst_t = _pad_edges(edge_index_topo[1])
    src_l = _pad_edges(edge_index_lc[0])
    dst_l = _pad_edges(edge_index_lc[1])
    tp, lp = topo_params, lc_params

    degp = _deg_k(dst_t, dst_l)
    dinv2 = _dinv_k(degp)
    dvt, dvl = dinv2[0], dinv2[1]

    hp1t = _mm1_k(xt, tp[0], dvt)
    hp1l = _mm1_k(xl, lp[0], dvl)
    S1 = _scat64(hp1t, src_t, dst_t, hp1l, src_l, dst_l)

    hp2t = _lay1t(S1, hp1t, dvt, tp[1], tp[2], tp[3], tp[4])
    hp2l = _lay1l(S1, hp1l, dvl, lp[1], lp[2], lp[3], lp[4])
    S2 = _scat64(hp2t, src_t, dst_t, hp2l, src_l, dst_l)

    hp3t = _lay2t(S2, hp2t, dvt, tp[5], tp[6], tp[7], tp[8])
    hp3l = _lay2l(S2, hp2l, dvl, lp[5], lp[6], lp[7], lp[8])
    S3 = _scat32(hp3t, src_t, dst_t, hp3l, src_l, dst_l)

    yt, lct = _fin_t(S3, hp3t, dvt, tp[9])
    yl, lcl = _fin_l(S3, hp3l, dvl, lp[9])

    pt, plc = _sortp(yt, lct, x_topo_batch, yl, lcl, x_lc_batch)
    mW1, mb1, mW2, mb2, mWo, mbo = mlp_params
    return _mlp_k(pt.reshape(NB, KP * 32), plc.reshape(NB, KP * 32),
                  mW1, mb1, mW2, mb2, mWo, mbo)


# speedup vs baseline: 18.9538x; 1.0485x over previous
import functools

import jax
import jax.numpy as jnp
from jax import lax
from jax.experimental import pallas as pl
from jax.experimental.pallas import tpu as pltpu
from jax.experimental.pallas import tpu_sc as plsc

NB = 16
KP = 30
NN = 10000
NE = 320000
DF = 128
NP = 10240
DUMMY = 10200
NC, NS, L = 2, 16, 16
CH = 128
EPT_P = 20480
EP = EPT_P * NS
BR = 1024


CHB = 512
NCHB = EPT_P // CHB


def _zero_acc(s, zb, acc, W):
    RT = NP // NS

    def zloop(i, _):
        r = i // (W // L)
        q = i % (W // L)
        zb[r, pl.ds(q * L, L)] = jnp.zeros((L,), jnp.float32)
        return 0
    lax.fori_loop(0, CH * (W // L), zloop, 0)
    for t in range(RT // CH):
        pltpu.sync_copy(zb, acc.at[pl.ds(s * RT + t * CH, CH)])


def _read_out(c, s, zb, acc, out):
    RT = NP // NS
    for t in range(RT // CH):
        r0 = s * RT + t * CH
        pltpu.sync_copy(acc.at[pl.ds(r0, CH)], zb)
        pltpu.sync_copy(zb, out.at[c, pl.ds(r0, CH)])


def _sc_scatter(W):
    mesh = plsc.VectorSubcoreMesh(core_axis_name="c", subcore_axis_name="s")

    def body(hp_t, e3_t, hp_l, e3_l, out,
             ea, eb, rowsa, rowsb, zb, acc, semg, sems):
        c = lax.axis_index("c")
        s = lax.axis_index("s")

        _zero_acc(s, zb, acc, W)
        plsc.subcore_barrier()

        def run(hp, e3):
            pltpu.sync_copy(e3.at[s * NCHB], ea)
            pltpu.async_copy(hp.at[ea.at[0]], rowsa, semg)

            def phase(i, cur_e, cur_r, nxt_e, nxt_r):
                @pl.when(i > 0)
                def _():
                    pltpu.make_async_copy(nxt_r, acc.at[nxt_e.at[1]],
                                          sems).wait()

                @pl.when(i + 1 < NCHB)
                def _():
                    pltpu.sync_copy(e3.at[s * NCHB + i + 1], nxt_e)
                    pltpu.async_copy(hp.at[nxt_e.at[0]], nxt_r, semg)
                pltpu.make_async_copy(hp.at[cur_e.at[0]], cur_r, semg).wait()
                pltpu.async_copy(cur_r, acc.at[cur_e.at[1]], sems, add=True)

            def sup(i, _):
                @pl.when(i % 2 == 0)
                def _():
                    phase(i, ea, rowsa, eb, rowsb)

                @pl.when(i % 2 == 1)
                def _():
                    phase(i, eb, rowsb, ea, rowsa)
                return 0
            lax.fori_loop(0, NCHB, sup, 0)
            pltpu.make_async_copy(rowsb, acc.at[eb.at[1]], sems).wait()

        @pl.when(c == 0)
        def _():
            run(hp_t, e3_t)

        @pl.when(c == 1)
        def _():
            run(hp_l, e3_l)

        plsc.subcore_barrier()
        _read_out(c, s, zb, acc, out)

    return pl.kernel(
        body,
        out_type=jax.ShapeDtypeStruct((NC, NP, W), jnp.float32),
        mesh=mesh,
        scratch_types=[
            pltpu.VMEM((2, CHB), jnp.int32),
            pltpu.VMEM((2, CHB), jnp.int32),
            pltpu.VMEM((CHB, W), jnp.float32),
            pltpu.VMEM((CHB, W), jnp.float32),
            pltpu.VMEM((CH, W), jnp.float32),
            pltpu.VMEM_SHARED((NP, W), jnp.float32),
            pltpu.SemaphoreType.DMA,
            pltpu.SemaphoreType.DMA,
        ],
        compiler_params=pltpu.CompilerParams(use_tc_tiling_on_sc=False, needs_layout_passes=False),
    )


def _sc_deg():
    W = 16
    KF = 4
    mesh = plsc.VectorSubcoreMesh(core_axis_name="c", subcore_axis_name="s")

    def body(e3_t, e3_l, out, e0, e1, e2, e3b, ones, zb, acc, sems):
        c = lax.axis_index("c")
        s = lax.axis_index("s")
        ebufs = [e0, e1, e2, e3b]

        def oloop(i, _):
            ones[i, pl.ds(0, L)] = jnp.full((L,), 1.0, jnp.float32)
            return 0
        lax.fori_loop(0, CHB, oloop, 0)
        _zero_acc(s, zb, acc, W)
        plsc.subcore_barrier()

        def run(e3):
            def sup(i, _):
                for b in range(KF):
                    pltpu.sync_copy(e3.at[s * NCHB + i * KF + b], ebufs[b])
                sc = [pltpu.async_copy(ones, acc.at[ebufs[b].at[1]],
                                       sems, add=True) for b in range(KF)]
                for d in sc:
                    d.wait()
                return 0
            lax.fori_loop(0, NCHB // KF, sup, 0)

        @pl.when(c == 0)
        def _():
            run(e3_t)

        @pl.when(c == 1)
        def _():
            run(e3_l)

        plsc.subcore_barrier()
        _read_out(c, s, zb, acc, out)

    return pl.kernel(
        body,
        out_type=jax.ShapeDtypeStruct((NC, NP, W), jnp.float32),
        mesh=mesh,
        scratch_types=[
            pltpu.VMEM((2, CHB), jnp.int32),
            pltpu.VMEM((2, CHB), jnp.int32),
            pltpu.VMEM((2, CHB), jnp.int32),
            pltpu.VMEM((2, CHB), jnp.int32),
            pltpu.VMEM((CHB, W), jnp.float32),
            pltpu.VMEM((CH, W), jnp.float32),
            pltpu.VMEM_SHARED((NP, W), jnp.float32),
            pltpu.SemaphoreType.DMA,
        ],
        compiler_params=pltpu.CompilerParams(use_tc_tiling_on_sc=False, needs_layout_passes=False),
    )


def _sortpool():
    NV = NN // L
    mesh = plsc.VectorSubcoreMesh(core_axis_name="c", subcore_axis_name="s")

    def body(y_t, lc_t, bt_t, y_l, lc_l, bt_l, out_t, out_l,
             btv, lcv, vals, posb, isel, rows, sem):
        c = lax.axis_index("c")
        s = lax.axis_index("s")
        lanes = lax.iota(jnp.int32, L)
        neg = jnp.full((L,), -jnp.inf, jnp.float32)

        def run(y, lc, bt, out):
            b = s
            pltpu.sync_copy(bt, btv)
            pltpu.sync_copy(lc.at[pl.ds(0, NN)], lcv)

            def pre(i, _):
                vals[pl.ds(i * L, L)] = neg
                posb[pl.ds(i * L, L)] = jnp.full((L,), DUMMY, jnp.int32)
                return 0
            lax.fori_loop(0, NP // L, pre, 0)
            isel[pl.ds(0, L)] = jnp.full((L,), DUMMY, jnp.int32)
            isel[pl.ds(L, L)] = jnp.full((L,), DUMMY, jnp.int32)

            def comp(k2, cnt):
                m = btv[pl.ds(k2 * L, L)] == b
                v = lcv[pl.ds(k2 * L, L)]
                pc = plsc.cumsum(m.astype(jnp.int32))
                idx = cnt + pc - 1
                plsc.store_scatter(vals, [idx], v, mask=m)
                plsc.store_scatter(posb, [idx], lanes + k2 * L, mask=m)
                return cnt + jnp.max(pc)
            cnt = lax.fori_loop(0, NV, comp, jnp.int32(0))
            nvec = (cnt + L - 1) // L

            def sel(t, _):
                mv = lax.fori_loop(
                    0, nvec,
                    lambda j, a: jnp.maximum(a, vals[pl.ds(j * L, L)]), neg)
                mx = jnp.max(mv)
                j = lax.while_loop(
                    lambda j: jnp.logical_not(
                        jnp.any(vals[pl.ds(j * L, L)] == mx)),
                    lambda j: j + 1, jnp.int32(0))
                v = vals[pl.ds(j * L, L)]
                eq = v == mx
                first = jnp.logical_and(eq, lanes == plsc.all_reduce_ffs(eq))
                pos = posb[pl.ds(j * L, L)]
                posx = jnp.where(mx == -jnp.inf,
                                 jnp.full((L,), DUMMY, jnp.int32), pos)
                plsc.store_scatter(isel, [jnp.full((L,), t, jnp.int32)],
                                   posx, mask=first)
                vals[pl.ds(j * L, L)] = jnp.where(first, neg, v)
                return 0
            lax.fori_loop(0, KP, sel, 0)

            pltpu.async_copy(y.at[isel], rows, sem).wait()
            pltpu.sync_copy(rows.at[pl.ds(0, KP)], out.at[b])

        @pl.when(c == 0)
        def _():
            run(y_t, lc_t, bt_t, out_t)

        @pl.when(c == 1)
        def _():
            run(y_l, lc_l, bt_l, out_l)

    return pl.kernel(
        body,
        out_type=(jax.ShapeDtypeStruct((NB, KP, 32), jnp.float32),
                  jax.ShapeDtypeStruct((NB, KP, 32), jnp.float32)),
        mesh=mesh,
        scratch_types=[
            pltpu.VMEM((NN,), jnp.int32),
            pltpu.VMEM((NN,), jnp.float32),
            pltpu.VMEM((NP,), jnp.float32),
            pltpu.VMEM((NP,), jnp.int32),
            pltpu.VMEM((2 * L,), jnp.int32),
            pltpu.VMEM((2 * L, 32), jnp.float32),
            pltpu.SemaphoreType.DMA,
        ],
        compiler_params=pltpu.CompilerParams(use_tc_tiling_on_sc=False, needs_layout_passes=False),
    )


def _dinv():
    def k(pref, oref):
        oref[...] = lax.rsqrt(pref[:, :, 0] + 1.0)
    return pl.pallas_call(
        k, grid=(NP // BR,),
        in_specs=[pl.BlockSpec((2, BR, 16), lambda i: (0, i, 0))],
        out_specs=pl.BlockSpec((2, BR), lambda i: (0, i)),
        out_shape=jax.ShapeDtypeStruct((2, NP), jnp.float32))


def _mm1():
    def k(xref, wref, dref, oref):
        h = jnp.dot(xref[...], wref[...], preferred_element_type=jnp.float32)
        oref[...] = dref[...][:, None] * h
    return pl.pallas_call(
        k, grid=(NP // BR,),
        in_specs=[pl.BlockSpec((BR, DF), lambda i: (i, 0)),
                  pl.BlockSpec((DF, 64), lambda i: (0, 0)),
                  pl.BlockSpec((BR,), lambda i: (i,))],
        out_specs=pl.BlockSpec((BR, 64), lambda i: (i, 0)),
        out_shape=jax.ShapeDtypeStruct((NP, 64), jnp.float32))


def _layer(b, W, Wo):
    def k(sref, href, dref, b1ref, lwref, lbref, wnref, oref):
        dv = dref[...][:, None]
        g = dv * (sref[0] + href[...]) + b1ref[...][None, :]
        x2 = jnp.where(g >= 0, g, 0.01 * g) + jnp.dot(
            g, lwref[...], preferred_element_type=jnp.float32) + lbref[...][None, :]
        oref[...] = dv * jnp.dot(x2, wnref[...],
                                 preferred_element_type=jnp.float32)
    return pl.pallas_call(
        k, grid=(NP // BR,),
        in_specs=[pl.BlockSpec((1, BR, W), lambda i: (b, i, 0)),
                  pl.BlockSpec((BR, W), lambda i: (i, 0)),
                  pl.BlockSpec((BR,), lambda i: (i,)),
                  pl.BlockSpec((W,), lambda i: (0,)),
                  pl.BlockSpec((W, W), lambda i: (0, 0)),
                  pl.BlockSpec((W,), lambda i: (0,)),
                  pl.BlockSpec((W, Wo), lambda i: (0, 0))],
        out_specs=pl.BlockSpec((BR, Wo), lambda i: (i, 0)),
        out_shape=jax.ShapeDtypeStruct((NP, Wo), jnp.float32))


def _final(b):
    def k(sref, href, dref, boref, yref, lref):
        dv = dref[...][:, None]
        g = dv * (sref[0] + href[...]) + boref[...][None, :]
        rid = lax.broadcasted_iota(jnp.int32, (BR, 1), 0) + pl.program_id(0) * BR
        y = jnp.where(rid < NN, g, 0.0)
        yref[...] = y
        lref[...] = y[:, 31]
    return pl.pallas_call(
        k, grid=(NP // BR,),
        in_specs=[pl.BlockSpec((1, BR, 32), lambda i: (b, i, 0)),
                  pl.BlockSpec((BR, 32), lambda i: (i, 0)),
                  pl.BlockSpec((BR,), lambda i: (i,)),
                  pl.BlockSpec((32,), lambda i: (0,))],
        out_specs=[pl.BlockSpec((BR, 32), lambda i: (i, 0)),
                   pl.BlockSpec((BR,), lambda i: (i,))],
        out_shape=[jax.ShapeDtypeStruct((NP, 32), jnp.float32),
                   jax.ShapeDtypeStruct((NP,), jnp.float32)])


def _mlp():
    def k(xt, xl, w1, b1, w2, b2, wo, bo, oref):
        w1f = w1[...]
        a = (jnp.dot(xt[...], w1f[:KP * 32], preferred_element_type=jnp.float32)
             + jnp.dot(xl[...], w1f[KP * 32:], preferred_element_type=jnp.float32)
             + b1[...][None, :])
        a = jnp.where(a >= 0, a, 0.01 * a)
        h = jnp.dot(a, w2[...], preferred_element_type=jnp.float32) + b2[...][None, :]
        h = jnp.where(h >= 0, h, 0.01 * h)
        oref[...] = jnp.dot(h, wo[...], preferred_element_type=jnp.float32) + bo[...][None, :]
    return pl.pallas_call(
        k, out_shape=jax.ShapeDtypeStruct((NB, 1), jnp.float32))


_deg_k = _sc_deg()
_scat64 = _sc_scatter(64)
_scat32 = _sc_scatter(32)
_sortp = _sortpool()
_dinv_k = _dinv()
_mm1_k = _mm1()
_lay1t = _layer(0, 64, 64)
_lay1l = _layer(1, 64, 64)
_lay2t = _layer(0, 64, 32)
_lay2l = _layer(1, 64, 32)
_fin_t = _final(0)
_fin_l = _final(1)
_mlp_k = _mlp()


def _pad_edges(ei):
    src = jnp.pad(ei[0], (0, EP - NE), constant_values=DUMMY)
    dst = jnp.pad(ei[1], (0, EP - NE), constant_values=DUMMY)
    return jnp.stack([src.reshape(NS * NCHB, CHB),
                      dst.reshape(NS * NCHB, CHB)], axis=1)


def kernel(x_topo, edge_index_topo, x_topo_batch, x_lc, edge_index_lc,
           x_lc_batch, topo_params, lc_params, mlp_params):
    f32 = jnp.float32
    xt = jnp.pad(x_topo.astype(f32), ((0, NP - NN), (0, 0)))
    xl = jnp.pad(x_lc.astype(f32), ((0, NP - NN), (0, 0)))
    e3_t = _pad_edges(edge_index_topo)
    e3_l = _pad_edges(edge_index_lc)
    tp, lp = topo_params, lc_params

    degp = _deg_k(e3_t, e3_l)
    dinv2 = _dinv_k(degp)
    dvt, dvl = dinv2[0], dinv2[1]

    hp1t = _mm1_k(xt, tp[0], dvt)
    hp1l = _mm1_k(xl, lp[0], dvl)
    S1 = _scat64(hp1t, e3_t, hp1l, e3_l)

    hp2t = _lay1t(S1, hp1t, dvt, tp[1], tp[2], tp[3], tp[4])
    hp2l = _lay1l(S1, hp1l, dvl, lp[1], lp[2], lp[3], lp[4])
    S2 = _scat64(hp2t, e3_t, hp2l, e3_l)

    hp3t = _lay2t(S2, hp2t, dvt, tp[5], tp[6], tp[7], tp[8])
    hp3l = _lay2l(S2, hp2l, dvl, lp[5], lp[6], lp[7], lp[8])
    S3 = _scat32(hp3t, e3_t, hp3l, e3_l)

    yt, lct = _fin_t(S3, hp3t, dvt, tp[9])
    yl, lcl = _fin_l(S3, hp3l, dvl, lp[9])

    pt, plc = _sortp(yt, lct, x_topo_batch, yl, lcl, x_lc_batch)
    mW1, mb1, mW2, mb2, mWo, mbo = mlp_params
    return _mlp_k(pt.reshape(NB, KP * 32), plc.reshape(NB, KP * 32),
                  mW1, mb1, mW2, mb2, mWo, mbo)
